# baseline (speedup 1.0000x reference)
import functools

import jax
import jax.numpy as jnp
from jax import lax
from jax.experimental import pallas as pl
from jax.experimental.pallas import tpu as pltpu
from jax.experimental.pallas import tpu_sc as plsc

P_REAL = 16 * 830
P_PAD = 13312
Q = 4096
QB = 256
SENTINEL = 1.0e7


CH = 128


def _tc_scan_body(c_ref, px_ref, py_ref, r_ref, len_ref, idx_ref):
    c = c_ref[...]
    cx = c[:, 0:1] * 640.0
    cy = c[:, 1:2] * 640.0
    rv = r_ref[...]
    big = jnp.int32(2**30)
    nq = QB // 8
    run_min = [jnp.full((8, CH), jnp.inf, jnp.float32) for _ in range(nq)]
    run_chunk = [jnp.zeros((8, CH), jnp.int32) for _ in range(nq)]
    run_max = [jnp.zeros((8, CH), jnp.float32) for _ in range(nq)]

    for cidx in range(0, P_PAD // CH, 2):
        px0 = px_ref[cidx]
        py0 = py_ref[cidx]
        px1 = px_ref[cidx + 1]
        py1 = py_ref[cidx + 1]
        for qs in range(nq):
            cxs = cx[qs * 8:(qs + 1) * 8]
            cys = cy[qs * 8:(qs + 1) * 8]
            dx0 = px0 - cxs
            dy0 = py0 - cys
            e20 = dx0 * dx0 + dy0 * dy0
            m0 = jnp.where(jnp.maximum(jnp.abs(dx0), jnp.abs(dy0)) < rv,
                           e20, 0.0)
            dx1 = px1 - cxs
            dy1 = py1 - cys
            e21 = dx1 * dx1 + dy1 * dy1
            m1 = jnp.where(jnp.maximum(jnp.abs(dx1), jnp.abs(dy1)) < rv,
                           e21, 0.0)
            cpair = jnp.where(e21 < e20, jnp.int32(cidx + 1), jnp.int32(cidx))
            epair = jnp.minimum(e20, e21)
            upd = epair < run_min[qs]
            run_chunk[qs] = jnp.where(upd, cpair, run_chunk[qs])
            run_min[qs] = jnp.minimum(run_min[qs], epair)
            run_max[qs] = jnp.maximum(run_max[qs], jnp.maximum(m0, m1))
    iota0 = lax.broadcasted_iota(jnp.int32, (8, CH), 1)
    min_e2 = jnp.concatenate(
        [jnp.min(m, axis=1, keepdims=True) for m in run_min], axis=0)
    amin = jnp.concatenate(
        [jnp.min(jnp.where(m == jnp.min(m, axis=1, keepdims=True),
                           c * jnp.int32(CH) + iota0, big),
                 axis=1, keepdims=True)
         for m, c in zip(run_min, run_chunk)], axis=0)
    amin = jnp.clip(amin, 1, P_REAL - 3)
    max_in = jnp.concatenate(
        [jnp.max(m, axis=1, keepdims=True) for m in run_max], axis=0)
    mind = jnp.sqrt(min_e2 + 1e-12)
    maxd = jnp.sqrt(max_in + 1e-12)
    len_ref[...] = jnp.sqrt(jnp.maximum(maxd * maxd - mind * mind, 0.0) + 1e-12)
    idx_ref[...] = amin


def _tc_scan(centers, px, py, r_arr):
    return pl.pallas_call(
        _tc_scan_body,
        grid=(Q // QB,),
        in_specs=[
            pl.BlockSpec((QB, 2), lambda i: (i, 0)),
            pl.BlockSpec((P_PAD // CH, 8, CH), lambda i: (0, 0, 0)),
            pl.BlockSpec((P_PAD // CH, 8, CH), lambda i: (0, 0, 0)),
            pl.BlockSpec((1, 1), lambda i: (0, 0)),
        ],
        out_specs=[
            pl.BlockSpec((QB, 1), lambda i: (i, 0)),
            pl.BlockSpec((QB, 1), lambda i: (i, 0)),
        ],
        out_shape=[
            jax.ShapeDtypeStruct((Q, 1), jnp.float32),
            jax.ShapeDtypeStruct((Q, 1), jnp.int32),
        ],
    )(centers, px, py, r_arr)


def _atan(x):
    t = jnp.abs(x)
    big = t > 2.414213562373095
    mid = t > 0.41421356237309503
    arg = jnp.where(big, 1.0 / t, jnp.where(mid, (t - 1.0) / (t + 1.0), t))
    z = arg * arg
    p = ((8.05374449538e-2 * z - 1.38776856032e-1) * z
         + 1.99777106478e-1) * z - 3.33329491539e-1
    rr = arg + arg * z * p
    half_pi = jnp.float32(1.5707963267948966)
    quarter_pi = jnp.float32(0.7853981633974483)
    y = jnp.where(big, half_pi - rr, jnp.where(mid, quarter_pi + rr, rr))
    return jnp.where(x < 0, -y, y)


_NC = 2
_NS = 16
_NW = _NC * _NS
_QW = Q // _NW
_L = 16


def _sc_finalize_body(px_hbm, py_hbm, idx_hbm, cx_hbm, cy_hbm, r_hbm,
                      conf_hbm, ox_hbm, oy_hbm, ang_hbm,
                      px_v, py_v, idx_v, cx_v, cy_v, r_v,
                      conf_v, ox_v, oy_v, ang_v):
    wid = lax.axis_index("s") * _NC + lax.axis_index("c")
    base = wid * _QW
    pltpu.sync_copy(px_hbm, px_v)
    pltpu.sync_copy(py_hbm, py_v)
    pltpu.sync_copy(idx_hbm.at[pl.ds(base, _QW)], idx_v)
    pltpu.sync_copy(cx_hbm.at[pl.ds(base, _QW)], cx_v)
    pltpu.sync_copy(cy_hbm.at[pl.ds(base, _QW)], cy_v)
    pltpu.sync_copy(r_hbm, r_v)
    rv = r_v[...]
    one = jnp.ones((_L,), jnp.int32)
    for j in range(_QW // _L):
        sl = pl.ds(j * _L, _L)
        idx = idx_v[sl]
        mx = plsc.load_gather(px_v, [idx]) * 640.0
        my = plsc.load_gather(py_v, [idx]) * 640.0
        bx = plsc.load_gather(px_v, [idx - one]) * 640.0
        by = plsc.load_gather(py_v, [idx - one]) * 640.0
        ax = plsc.load_gather(px_v, [idx + one]) * 640.0
        ay = plsc.load_gather(py_v, [idx + one]) * 640.0
        cxs = cx_v[sl] * 640.0
        cys = cy_v[sl] * 640.0
        conf = (jnp.abs(mx - cxs) < rv) & (jnp.abs(my - cys) < rv)
        conf_v[sl] = jnp.where(conf, jnp.int32(1), jnp.int32(0))
        ox_v[sl] = (mx - (cxs - rv)) / rv / 2.0
        oy_v[sl] = (my - (cys - rv)) / rv / 2.0
        scope = (ay - my) / (ax - mx + 1e-12)
        scope = scope + (my - by) / (mx - bx + 1e-12)
        scope = scope / 2.0
        ang_v[sl] = _atan(scope)
    pltpu.sync_copy(conf_v, conf_hbm.at[pl.ds(base, _QW)])
    pltpu.sync_copy(ox_v, ox_hbm.at[pl.ds(base, _QW)])
    pltpu.sync_copy(oy_v, oy_hbm.at[pl.ds(base, _QW)])
    pltpu.sync_copy(ang_v, ang_hbm.at[pl.ds(base, _QW)])


@functools.lru_cache(maxsize=1)
def _sc_finalize():
    return pl.kernel(
        _sc_finalize_body,
        mesh=plsc.VectorSubcoreMesh(core_axis_name="c", subcore_axis_name="s"),
        compiler_params=pltpu.CompilerParams(needs_layout_passes=False),
        out_type=[
            jax.ShapeDtypeStruct((Q,), jnp.int32),
            jax.ShapeDtypeStruct((Q,), jnp.float32),
            jax.ShapeDtypeStruct((Q,), jnp.float32),
            jax.ShapeDtypeStruct((Q,), jnp.float32),
        ],
        scratch_types=[
            pltpu.VMEM((P_PAD,), jnp.float32),
            pltpu.VMEM((P_PAD,), jnp.float32),
            pltpu.VMEM((_QW,), jnp.int32),
            pltpu.VMEM((_QW,), jnp.float32),
            pltpu.VMEM((_QW,), jnp.float32),
            pltpu.VMEM((_L,), jnp.float32),
            pltpu.VMEM((_QW,), jnp.int32),
            pltpu.VMEM((_QW,), jnp.float32),
            pltpu.VMEM((_QW,), jnp.float32),
            pltpu.VMEM((_QW,), jnp.float32),
        ],
    )


def kernel(target_points, img_centers, r):
    pts = target_points.reshape(-1, 2)
    pad = P_PAD - P_REAL
    px = jnp.pad(pts[:, 0], (0, pad), constant_values=SENTINEL)
    py = jnp.pad(pts[:, 1], (0, pad), constant_values=SENTINEL)
    rf = jnp.asarray(r, jnp.float32)
    r_arr = rf.reshape(1, 1)
    nch = P_PAD // CH
    px_rep = jnp.broadcast_to((px * 640.0).reshape(nch, 1, CH), (nch, 8, CH))
    py_rep = jnp.broadcast_to((py * 640.0).reshape(nch, 1, CH), (nch, 8, CH))
    gt_length, amin = _tc_scan(img_centers, px_rep, py_rep, r_arr)
    conf_i, ox, oy, gt_angle = _sc_finalize()(
        px, py, amin.reshape(Q),
        img_centers[:, 0], img_centers[:, 1],
        jnp.full((_L,), rf, jnp.float32))
    gt_confidence = conf_i != 0
    offset = jnp.stack([ox, oy], axis=-1)
    return (gt_confidence, offset, gt_angle, gt_length.reshape(Q))


# speedup vs baseline: 1.0175x; 1.0175x over previous
import functools

import jax
import jax.numpy as jnp
from jax import lax
from jax.experimental import pallas as pl
from jax.experimental.pallas import tpu as pltpu
from jax.experimental.pallas import tpu_sc as plsc

P_REAL = 16 * 830
P_PAD = 13312
Q = 4096
QB = 256
SENTINEL = 1.0e7


CH = 128


def _tc_scan_body(c_ref, px_ref, py_ref, r_ref, len_ref, idx_ref):
    c = c_ref[...]
    cx = c[:, 0:1] * 640.0
    cy = c[:, 1:2] * 640.0
    rv = r_ref[...]
    big = jnp.int32(2**30)
    nq = QB // 8
    run_min = [jnp.full((8, CH), jnp.inf, jnp.float32) for _ in range(nq)]
    run_chunk = [jnp.zeros((8, CH), jnp.int32) for _ in range(nq)]
    run_max = [jnp.zeros((8, CH), jnp.float32) for _ in range(nq)]

    for cidx in range(P_PAD // CH):
        px = px_ref[cidx]
        py = py_ref[cidx]
        for qs in range(nq):
            dx = px - cx[qs * 8:(qs + 1) * 8]
            dy = py - cy[qs * 8:(qs + 1) * 8]
            e2 = dx * dx + dy * dy
            upd = e2 < run_min[qs]
            run_chunk[qs] = jnp.where(upd, jnp.int32(cidx), run_chunk[qs])
            run_min[qs] = jnp.minimum(run_min[qs], e2)
            cheb = jnp.maximum(jnp.abs(dx), jnp.abs(dy))
            run_max[qs] = jnp.maximum(run_max[qs],
                                      jnp.where(cheb < rv, e2, 0.0))
    iota0 = lax.broadcasted_iota(jnp.int32, (8, CH), 1)
    min_e2 = jnp.concatenate(
        [jnp.min(m, axis=1, keepdims=True) for m in run_min], axis=0)
    amin = jnp.concatenate(
        [jnp.min(jnp.where(m == jnp.min(m, axis=1, keepdims=True),
                           c * jnp.int32(CH) + iota0, big),
                 axis=1, keepdims=True)
         for m, c in zip(run_min, run_chunk)], axis=0)
    amin = jnp.clip(amin, 1, P_REAL - 3)
    max_in = jnp.concatenate(
        [jnp.max(m, axis=1, keepdims=True) for m in run_max], axis=0)
    mind = jnp.sqrt(min_e2 + 1e-12)
    maxd = jnp.sqrt(max_in + 1e-12)
    len_ref[...] = jnp.sqrt(jnp.maximum(maxd * maxd - mind * mind, 0.0) + 1e-12)
    idx_ref[...] = amin


def _tc_scan(centers, px, py, r_arr):
    return pl.pallas_call(
        _tc_scan_body,
        grid=(Q // QB,),
        in_specs=[
            pl.BlockSpec((QB, 2), lambda i: (i, 0)),
            pl.BlockSpec((P_PAD // CH, 8, CH), lambda i: (0, 0, 0)),
            pl.BlockSpec((P_PAD // CH, 8, CH), lambda i: (0, 0, 0)),
            pl.BlockSpec((1, 1), lambda i: (0, 0)),
        ],
        out_specs=[
            pl.BlockSpec((QB, 1), lambda i: (i, 0)),
            pl.BlockSpec((QB, 1), lambda i: (i, 0)),
        ],
        out_shape=[
            jax.ShapeDtypeStruct((Q, 1), jnp.float32),
            jax.ShapeDtypeStruct((Q, 1), jnp.int32),
        ],
    )(centers, px, py, r_arr)


def _atan(x):
    t = jnp.abs(x)
    big = t > 2.414213562373095
    mid = t > 0.41421356237309503
    arg = jnp.where(big, 1.0 / t, jnp.where(mid, (t - 1.0) / (t + 1.0), t))
    z = arg * arg
    p = ((8.05374449538e-2 * z - 1.38776856032e-1) * z
         + 1.99777106478e-1) * z - 3.33329491539e-1
    rr = arg + arg * z * p
    half_pi = jnp.float32(1.5707963267948966)
    quarter_pi = jnp.float32(0.7853981633974483)
    y = jnp.where(big, half_pi - rr, jnp.where(mid, quarter_pi + rr, rr))
    return jnp.where(x < 0, -y, y)


_NC = 2
_NS = 16
_NW = _NC * _NS
_QW = Q // _NW
_L = 16


def _sc_finalize_body(px_hbm, py_hbm, idx_hbm, cx_hbm, cy_hbm, r_hbm,
                      conf_hbm, ox_hbm, oy_hbm, ang_hbm,
                      px_v, py_v, idx_v, cx_v, cy_v, r_v,
                      conf_v, ox_v, oy_v, ang_v):
    wid = lax.axis_index("s") * _NC + lax.axis_index("c")
    base = wid * _QW
    pltpu.sync_copy(px_hbm, px_v)
    pltpu.sync_copy(py_hbm, py_v)
    pltpu.sync_copy(idx_hbm.at[pl.ds(base, _QW)], idx_v)
    pltpu.sync_copy(cx_hbm.at[pl.ds(base, _QW)], cx_v)
    pltpu.sync_copy(cy_hbm.at[pl.ds(base, _QW)], cy_v)
    pltpu.sync_copy(r_hbm, r_v)
    rv = r_v[...]
    one = jnp.ones((_L,), jnp.int32)
    for j in range(_QW // _L):
        sl = pl.ds(j * _L, _L)
        idx = idx_v[sl]
        mx = plsc.load_gather(px_v, [idx]) * 640.0
        my = plsc.load_gather(py_v, [idx]) * 640.0
        bx = plsc.load_gather(px_v, [idx - one]) * 640.0
        by = plsc.load_gather(py_v, [idx - one]) * 640.0
        ax = plsc.load_gather(px_v, [idx + one]) * 640.0
        ay = plsc.load_gather(py_v, [idx + one]) * 640.0
        cxs = cx_v[sl] * 640.0
        cys = cy_v[sl] * 640.0
        conf = (jnp.abs(mx - cxs) < rv) & (jnp.abs(my - cys) < rv)
        conf_v[sl] = jnp.where(conf, jnp.int32(1), jnp.int32(0))
        ox_v[sl] = (mx - (cxs - rv)) / rv / 2.0
        oy_v[sl] = (my - (cys - rv)) / rv / 2.0
        scope = (ay - my) / (ax - mx + 1e-12)
        scope = scope + (my - by) / (mx - bx + 1e-12)
        scope = scope / 2.0
        ang_v[sl] = _atan(scope)
    pltpu.sync_copy(conf_v, conf_hbm.at[pl.ds(base, _QW)])
    pltpu.sync_copy(ox_v, ox_hbm.at[pl.ds(base, _QW)])
    pltpu.sync_copy(oy_v, oy_hbm.at[pl.ds(base, _QW)])
    pltpu.sync_copy(ang_v, ang_hbm.at[pl.ds(base, _QW)])


@functools.lru_cache(maxsize=1)
def _sc_finalize():
    return pl.kernel(
        _sc_finalize_body,
        mesh=plsc.VectorSubcoreMesh(core_axis_name="c", subcore_axis_name="s"),
        compiler_params=pltpu.CompilerParams(needs_layout_passes=False),
        out_type=[
            jax.ShapeDtypeStruct((Q,), jnp.int32),
            jax.ShapeDtypeStruct((Q,), jnp.float32),
            jax.ShapeDtypeStruct((Q,), jnp.float32),
            jax.ShapeDtypeStruct((Q,), jnp.float32),
        ],
        scratch_types=[
            pltpu.VMEM((P_PAD,), jnp.float32),
            pltpu.VMEM((P_PAD,), jnp.float32),
            pltpu.VMEM((_QW,), jnp.int32),
            pltpu.VMEM((_QW,), jnp.float32),
            pltpu.VMEM((_QW,), jnp.float32),
            pltpu.VMEM((_L,), jnp.float32),
            pltpu.VMEM((_QW,), jnp.int32),
            pltpu.VMEM((_QW,), jnp.float32),
            pltpu.VMEM((_QW,), jnp.float32),
            pltpu.VMEM((_QW,), jnp.float32),
        ],
    )


def kernel(target_points, img_centers, r):
    pts = target_points.reshape(-1, 2)
    pad = P_PAD - P_REAL
    px = jnp.pad(pts[:, 0], (0, pad), constant_values=SENTINEL)
    py = jnp.pad(pts[:, 1], (0, pad), constant_values=SENTINEL)
    rf = jnp.asarray(r, jnp.float32)
    r_arr = rf.reshape(1, 1)
    nch = P_PAD // CH
    px_rep = jnp.broadcast_to((px * 640.0).reshape(nch, 1, CH), (nch, 8, CH))
    py_rep = jnp.broadcast_to((py * 640.0).reshape(nch, 1, CH), (nch, 8, CH))
    gt_length, amin = _tc_scan(img_centers, px_rep, py_rep, r_arr)
    conf_i, ox, oy, gt_angle = _sc_finalize()(
        px, py, amin.reshape(Q),
        img_centers[:, 0], img_centers[:, 1],
        jnp.full((_L,), rf, jnp.float32))
    gt_confidence = conf_i != 0
    offset = jnp.stack([ox, oy], axis=-1)
    return (gt_confidence, offset, gt_angle, gt_length.reshape(Q))
